# baseline (speedup 1.0000x reference)
import functools

import jax
import jax.numpy as jnp
import numpy as np
from jax.experimental import pallas as pl
from jax.experimental.pallas import tpu as pltpu


def _bilinear_matrix(out_size: int, in_size: int) -> np.ndarray:
    W = np.zeros((out_size, in_size), dtype=np.float32)
    scale = in_size / out_size
    for i in range(out_size):
        src = (i + 0.5) * scale - 0.5
        src = max(src, 0.0)
        i0 = int(np.floor(src))
        i0 = min(i0, in_size - 1)
        i1 = min(i0 + 1, in_size - 1)
        lam = src - i0
        W[i, i0] += 1.0 - lam
        W[i, i1] += lam
    return W


@functools.lru_cache(maxsize=None)
def _folded_interp_matrices(out_h: int, out_w: int, in_h: int, in_w: int, p: int):
    ww = _bilinear_matrix(out_w, in_w)
    wh = _bilinear_matrix(out_h, in_h)
    wp = np.zeros((p * in_w, p * out_w), dtype=np.float32)
    for r in range(p):
        wp[r * in_w:(r + 1) * in_w, r * out_w:(r + 1) * out_w] = ww.T
    rows = in_h // p
    perm = [q * p + r for r in range(p) for q in range(rows)]
    whp = np.ascontiguousarray(wh[:, perm])
    return jnp.asarray(whp), jnp.asarray(wp)


def _fused_kernel(ft_ref, fs_ref, whp_ref, wp_ref, out_ref, *, p, Wout):
    eps = 1e-12
    ft = ft_ref[0].astype(jnp.float32)
    fs = fs_ref[0].astype(jnp.float32)

    s_tt = jnp.sum(ft * ft, axis=0)
    s_ss = jnp.sum(fs * fs, axis=0)
    s_ts = jnp.sum(ft * fs, axis=0)

    inv_t = 1.0 / jnp.maximum(jnp.sqrt(s_tt), eps)
    inv_s = 1.0 / jnp.maximum(jnp.sqrt(s_ss), eps)
    lmf = 0.5 * (s_tt * inv_t * inv_t + s_ss * inv_s * inv_s) - s_ts * (inv_t * inv_s)

    tmpf = jnp.dot(lmf, wp_ref[...], preferred_element_type=jnp.float32)
    tmpc = jnp.concatenate(
        [tmpf[:, r * Wout:(r + 1) * Wout] for r in range(p)], axis=0)
    out = jnp.dot(whp_ref[...], tmpc, preferred_element_type=jnp.float32)
    out_ref[0, 0] = out


@jax.jit
def _forward(ft, fs, whp, wp):
    B, C, Hf, Wf = ft.shape
    Hout = whp.shape[0]
    p = wp.shape[0] // Wf
    Wout = wp.shape[1] // p
    rows = Hf // p

    ftf = ft.reshape(B, C, rows, p * Wf)
    fsf = fs.reshape(B, C, rows, p * Wf)

    HW = Hf * Wf
    itemsize = jnp.dtype(ft.dtype).itemsize
    cost = pl.CostEstimate(
        flops=int(B * (6 * C * HW + 12 * HW)
                  + 2 * B * (rows * p * Wf * p * Wout + Hout * Hf * Wout)),
        transcendentals=int(2 * B * HW),
        bytes_accessed=int(2 * B * C * HW * itemsize + B * Hout * Wout * 4),
    )
    out = pl.pallas_call(
        functools.partial(_fused_kernel, p=p, Wout=Wout),
        out_shape=jax.ShapeDtypeStruct((B, 1, Hout, Wout), jnp.float32),
        grid=(B,),
        in_specs=[
            pl.BlockSpec((1, C, rows, p * Wf), lambda b: (b, 0, 0, 0)),
            pl.BlockSpec((1, C, rows, p * Wf), lambda b: (b, 0, 0, 0)),
            pl.BlockSpec((Hout, Hf), lambda b: (0, 0)),
            pl.BlockSpec((p * Wf, p * Wout), lambda b: (0, 0)),
        ],
        out_specs=pl.BlockSpec((1, 1, Hout, Wout), lambda b: (b, 0, 0, 0)),
        compiler_params=pltpu.CompilerParams(
            dimension_semantics=("parallel",),
            vmem_limit_bytes=96 << 20,
        ),
        cost_estimate=cost,
    )(ftf, fsf, whp, wp)
    return out


def kernel(ft, fs):
    img_size = (32, 3, 256, 256)
    _, _, out_h, out_w = img_size
    _, _, Hf, Wf = ft.shape
    p = 128 // Wf if (Wf <= 128 and 128 % Wf == 0) else 1
    while Hf % p:
        p //= 2
    whp, wp = _folded_interp_matrices(int(out_h), int(out_w), int(Hf), int(Wf), p)
    return _forward(ft, fs, whp, wp)
